# baseline (speedup 1.0000x reference)
import functools

import numpy as np
import jax
import jax.numpy as jnp
from jax import lax
from jax.experimental import pallas as pl
from jax.experimental.pallas import tpu as pltpu
from jax.experimental.pallas import tpu_sc as plsc

_OC, _IC, _ORD, _K = 384, 192, 4, 5
_ROW = _ORD * _K * _K
_NR_IN = _IC * _K * _K * _ORD
_NR_OUT = 4 * _NR_IN
_NW = 32
_RPW = _NR_OUT // _NW
_CH = 100
_NCH = _RPW // _CH


def _row_table() -> np.ndarray:
    a = np.arange(_ROW).reshape(_ORD, _K, _K)
    perms = [np.rot90(np.roll(a, shift=r, axis=0), k=r, axes=(-2, -1)).reshape(_ROW)
             for r in range(4)]
    r_, ic_, y_, x_, h_ = np.meshgrid(
        np.arange(4), np.arange(_IC), np.arange(_K), np.arange(_K),
        np.arange(_ORD), indexing="ij")
    j = (h_ * _K + y_) * _K + x_
    src = np.stack(perms)[r_.ravel(), j.ravel()].reshape(j.shape)
    hp = src // (_K * _K)
    yp = (src % (_K * _K)) // _K
    xp = src % _K
    tab = (((ic_ * _K + yp) * _K + xp) * _ORD + hp).reshape(-1)
    return tab.astype(np.int32).reshape(_NW, _NCH, _CH)


_TAB = _row_table()

_MESH = plsc.VectorSubcoreMesh(core_axis_name="c", subcore_axis_name="s",
                               num_cores=2, num_subcores=16)


@functools.partial(
    pl.kernel,
    out_type=jax.ShapeDtypeStruct((_NR_OUT // _ORD, _ORD, _OC), jnp.float32),
    mesh=_MESH,
    scratch_types=[
        pltpu.VMEM((_NCH, _CH), jnp.int32),
        pltpu.VMEM((3, _CH, _OC), jnp.float32),
        pltpu.SemaphoreType.DMA,
        pltpu.SemaphoreType.DMA,
        pltpu.SemaphoreType.DMA,
        pltpu.SemaphoreType.DMA,
        pltpu.SemaphoreType.DMA,
        pltpu.SemaphoreType.DMA,
    ],
    compiler_params=pltpu.CompilerParams(needs_layout_passes=False),
)
def _bank(in3_hbm, tab_hbm, out3_hbm, idx_v, buf, *sems):
    in_hbm = in3_hbm.reshape(_NR_IN, _OC)
    out_hbm = out3_hbm.reshape(_NR_OUT, _OC)
    wid = lax.axis_index("s") * 2 + lax.axis_index("c")
    base = wid * _RPW
    pltpu.sync_copy(tab_hbm.at[wid], idx_v)
    gsems = sems[:3]
    osems = sems[3:]

    def gather(c):
        return pltpu.async_copy(in_hbm.at[idx_v.at[c]], buf.at[c % 3],
                                gsems[c % 3])

    def put(c):
        return pltpu.async_copy(buf.at[c % 3],
                                out_hbm.at[pl.ds(base + c * _CH, _CH)],
                                osems[c % 3])

    gh = {0: gather(0), 1: gather(1)}
    oh = {}
    for c in range(_NCH):
        gh[c].wait()
        oh[c] = put(c)
        n = c + 2
        if n < _NCH:
            if n - 3 >= 0:
                oh[n - 3].wait()
            gh[n] = gather(n)
    for c in range(_NCH - 3, _NCH):
        oh[c].wait()


def kernel(weight):
    in3 = weight.transpose(1, 3, 4, 2, 0).reshape(_NR_IN // _ORD, _ORD, _OC)
    out3 = _bank(in3, jnp.asarray(_TAB))
    out6 = out3.reshape(4, _IC, _K, _K, _ORD, _OC)
    return out6.transpose(5, 0, 1, 4, 2, 3)


# speedup vs baseline: 1.0129x; 1.0129x over previous
import functools

import numpy as np
import jax
import jax.numpy as jnp
from jax import lax
from jax.experimental import pallas as pl
from jax.experimental.pallas import tpu as pltpu
from jax.experimental.pallas import tpu_sc as plsc

_OC, _IC, _ORD, _K = 384, 192, 4, 5
_ROW = _ORD * _K * _K
_NR_IN = _IC * _K * _K * _ORD
_NR_OUT = 4 * _NR_IN
_NW = 32
_RPW = _NR_OUT // _NW
_CH = 96
_NCH = _RPW // _CH


def _row_table() -> np.ndarray:
    a = np.arange(_ROW).reshape(_ORD, _K, _K)
    perms = [np.rot90(np.roll(a, shift=r, axis=0), k=r, axes=(-2, -1)).reshape(_ROW)
             for r in range(4)]
    r_, ic_, y_, x_, h_ = np.meshgrid(
        np.arange(4), np.arange(_IC), np.arange(_K), np.arange(_K),
        np.arange(_ORD), indexing="ij")
    j = (h_ * _K + y_) * _K + x_
    src = np.stack(perms)[r_.ravel(), j.ravel()].reshape(j.shape)
    hp = src // (_K * _K)
    yp = (src % (_K * _K)) // _K
    xp = src % _K
    tab = (((ic_ * _K + yp) * _K + xp) * _ORD + hp).reshape(-1)
    return tab.astype(np.int32).reshape(_NW, _NCH, _CH)


_TAB = _row_table()

_MESH = plsc.VectorSubcoreMesh(core_axis_name="c", subcore_axis_name="s",
                               num_cores=2, num_subcores=16)


@functools.partial(
    pl.kernel,
    out_type=jax.ShapeDtypeStruct((_NR_OUT // _ORD, _ORD, _OC), jnp.float32),
    mesh=_MESH,
    scratch_types=[
        pltpu.VMEM((_NCH, _CH), jnp.int32),
        pltpu.VMEM((3, _CH, _OC), jnp.float32),
        pltpu.SemaphoreType.DMA,
        pltpu.SemaphoreType.DMA,
        pltpu.SemaphoreType.DMA,
        pltpu.SemaphoreType.DMA,
        pltpu.SemaphoreType.DMA,
        pltpu.SemaphoreType.DMA,
    ],
    compiler_params=pltpu.CompilerParams(needs_layout_passes=False),
)
def _bank(in3_hbm, tab_hbm, out3_hbm, idx_v, buf, *sems):
    in_hbm = in3_hbm.reshape(_NR_IN, _OC)
    out_hbm = out3_hbm.reshape(_NR_OUT, _OC)
    wid = lax.axis_index("s") * 2 + lax.axis_index("c")
    base = wid * _RPW
    pltpu.sync_copy(tab_hbm.at[wid], idx_v)
    gsems = sems[:3]
    osems = sems[3:]

    def gather(c):
        return pltpu.async_copy(in_hbm.at[idx_v.at[c]], buf.at[c % 3],
                                gsems[c % 3])

    def put(c):
        return pltpu.async_copy(buf.at[c % 3],
                                out_hbm.at[pl.ds(base + c * _CH, _CH)],
                                osems[c % 3])

    gh = {0: gather(0), 1: gather(1)}
    oh = {}
    for c in range(_NCH):
        gh[c].wait()
        oh[c] = put(c)
        n = c + 2
        if n < _NCH:
            if n - 3 >= 0:
                oh[n - 3].wait()
            gh[n] = gather(n)
    for c in range(_NCH - 3, _NCH):
        oh[c].wait()


def kernel(weight):
    in3 = weight.transpose(1, 3, 4, 2, 0).reshape(_NR_IN // _ORD, _ORD, _OC)
    out3 = _bank(in3, jnp.asarray(_TAB))
    out6 = out3.reshape(4, _IC, _K, _K, _ORD, _OC)
    return out6.transpose(5, 0, 1, 4, 2, 3)
